# Initial kernel scaffold; baseline (speedup 1.0000x reference)
import jax
import jax.numpy as jnp
from jax.experimental import pallas as pl


def kernel(x, edge_index, edge_attr, W1, root1, b1, W2, root2, b2):
    raise NotImplementedError("write your pallas kernel here")


# speedup vs baseline: 1.5566x; 1.5566x over previous
import jax
import jax.numpy as jnp
from jax import lax
from jax.experimental import pallas as pl
from jax.experimental.pallas import tpu as pltpu
from jax.experimental.pallas import tpu_sc as plsc

N = 10000
E = 160000
IN = 128
OUT = 128
KS = 5
K = KS * KS
NC = 2
NS = 16
NT = NC * NS
EPW = 5120
EPAD = NT * EPW
C = 64
NCH = EPW // C
CD = 320
NCHD = EPW // CD
NROW = 624
RB = 400
NRB = N // RB


def _basis_body(src_ref, p0_ref, p1_ref, w_ref, i_ref):
    src = src_ref[...]
    v0 = p0_ref[...] * (KS - 1.0)
    v1 = p1_ref[...] * (KS - 1.0)
    b0 = jnp.clip(jnp.floor(v0).astype(jnp.int32), 0, KS - 2)
    b1 = jnp.clip(jnp.floor(v1).astype(jnp.int32), 0, KS - 2)
    f0 = v0 - b0.astype(jnp.float32)
    f1 = v1 - b1.astype(jnp.float32)
    base = src * K + b0 + b1 * KS
    w_ref[0] = (1.0 - f0) * (1.0 - f1)
    w_ref[1] = f0 * (1.0 - f1)
    w_ref[2] = (1.0 - f0) * f1
    w_ref[3] = f0 * f1
    i_ref[0] = base
    i_ref[1] = base + 1
    i_ref[2] = base + KS
    i_ref[3] = base + KS + 1


def _basis(src2d, p02d, p12d):
    r = E // 128
    return pl.pallas_call(
        _basis_body,
        out_shape=[jax.ShapeDtypeStruct((4, r, 128), jnp.float32),
                   jax.ShapeDtypeStruct((4, r, 128), jnp.int32)],
    )(src2d, p02d, p12d)


def _z_body(x_ref, wt_ref, z_ref):
    z_ref[...] = jnp.dot(x_ref[...], wt_ref[...],
                         preferred_element_type=jnp.float32)


def _zmat(xx, wt):
    return pl.pallas_call(
        _z_body,
        grid=(NRB,),
        in_specs=[pl.BlockSpec((RB, IN), lambda i: (i, 0)),
                  pl.BlockSpec((IN, K * OUT), lambda i: (0, 0))],
        out_specs=pl.BlockSpec((RB, K * OUT), lambda i: (i, 0)),
        out_shape=jax.ShapeDtypeStruct((N, K * OUT), jnp.float32),
    )(xx, wt)


_sc_mesh = plsc.VectorSubcoreMesh(core_axis_name="c", subcore_axis_name="s",
                                  num_cores=NC, num_subcores=NS)


def _lane_splat(vec, l):
    idx = jnp.full((16,), l, dtype=jnp.int32)
    return lax.gather(
        vec, idx[:, None],
        dimension_numbers=lax.GatherDimensionNumbers(
            offset_dims=(), collapsed_slice_dims=(0,), start_index_map=(0,)),
        slice_sizes=(1,), mode=lax.GatherScatterMode.PROMISE_IN_BOUNDS)


def _sc_body(z_ref, im_ref, wm_ref, dm_ref, zero2_ref,
             acc_out,
             acc,
             ib0, ib1, ib2, ib3, wb0, wb1, wb2, wb3, db,
             gb0, gb1, gb2, gb3, mb):
    c = lax.axis_index("c")
    s = lax.axis_index("s")
    t = c * NS + s
    base = s * NROW
    tail = NS * NROW

    pltpu.sync_copy(zero2_ref.at[pl.ds(base, NROW)], acc.at[pl.ds(base, NROW)])

    @pl.when(s == NS - 1)
    def _():
        pltpu.sync_copy(zero2_ref.at[pl.ds(tail, N - tail)],
                        acc.at[pl.ds(tail, N - tail)])

    plsc.subcore_barrier()

    ibs = (ib0, ib1, ib2, ib3)
    wbs = (wb0, wb1, wb2, wb3)
    gbs = (gb0, gb1, gb2, gb3)

    def chunk(j, carry):
        mbase = t * EPW + j * C
        for q in range(4):
            pltpu.sync_copy(im_ref.at[pl.ds(q * EPAD + mbase, C)], ibs[q])
            pltpu.sync_copy(wm_ref.at[pl.ds(q * EPAD + mbase, C)], wbs[q])
        pltpu.sync_copy(dm_ref.at[pl.ds(mbase, C)], db)
        for q in range(4):
            pltpu.sync_copy(z_ref.at[ibs[q]], gbs[q])

        def grp(g, carry2):
            wv0 = wb0[pl.ds(g * 16, 16)]
            wv1 = wb1[pl.ds(g * 16, 16)]
            wv2 = wb2[pl.ds(g * 16, 16)]
            wv3 = wb3[pl.ds(g * 16, 16)]

            def lane(l, carry3):
                e = g * 16 + l
                s0 = _lane_splat(wv0, l)
                s1 = _lane_splat(wv1, l)
                s2 = _lane_splat(wv2, l)
                s3 = _lane_splat(wv3, l)
                for v in range(8):
                    col = pl.ds(v * 16, 16)
                    m = (gb0[e, col] * s0 + gb1[e, col] * s1 +
                         gb2[e, col] * s2 + gb3[e, col] * s3)
                    mb[e, col] = m
                return 0

            lax.fori_loop(0, 16, lane, 0)
            return 0

        lax.fori_loop(0, C // 16, grp, 0)
        pltpu.sync_copy(mb, acc.at[db], add=True)
        return 0

    lax.fori_loop(0, NCH, chunk, 0)
    plsc.subcore_barrier()
    pltpu.sync_copy(acc.at[pl.ds(base, NROW)],
                    acc_out.at[c, pl.ds(base, NROW)])

    @pl.when(s == NS - 1)
    def _():
        pltpu.sync_copy(acc.at[pl.ds(tail, N - tail)],
                        acc_out.at[c, pl.ds(tail, N - tail)])


_sc_gather_scatter = pl.kernel(
    _sc_body,
    out_type=jax.ShapeDtypeStruct((NC, N, OUT), jnp.float32),
    mesh=_sc_mesh,
    compiler_params=pltpu.CompilerParams(needs_layout_passes=False),
    scratch_types=[
        pltpu.VMEM_SHARED((N, OUT), jnp.float32),
        pltpu.VMEM((C,), jnp.int32),
        pltpu.VMEM((C,), jnp.int32),
        pltpu.VMEM((C,), jnp.int32),
        pltpu.VMEM((C,), jnp.int32),
        pltpu.VMEM((C,), jnp.float32),
        pltpu.VMEM((C,), jnp.float32),
        pltpu.VMEM((C,), jnp.float32),
        pltpu.VMEM((C,), jnp.float32),
        pltpu.VMEM((C,), jnp.int32),
        pltpu.VMEM((C, OUT), jnp.float32),
        pltpu.VMEM((C, OUT), jnp.float32),
        pltpu.VMEM((C, OUT), jnp.float32),
        pltpu.VMEM((C, OUT), jnp.float32),
        pltpu.VMEM((C, OUT), jnp.float32),
    ],
)


def _sc_deg_body(wm_ref, dm_ref, zero1_ref,
                 deg_out,
                 degbuf, wb0, wb1, wb2, wb3, db):
    c = lax.axis_index("c")
    s = lax.axis_index("s")
    t = c * NS + s

    pltpu.sync_copy(zero1_ref, degbuf)
    wbs = (wb0, wb1, wb2, wb3)

    def chunk(j, carry):
        mbase = t * EPW + j * CD
        for q in range(4):
            pltpu.sync_copy(wm_ref.at[pl.ds(q * EPAD + mbase, CD)], wbs[q])
        pltpu.sync_copy(dm_ref.at[pl.ds(mbase, CD)], db)

        def grp(g, carry2):
            sl = pl.ds(g * 16, 16)
            wsum = wb0[sl] + wb1[sl] + wb2[sl] + wb3[sl]
            plsc.addupdate_scatter(degbuf, [db[sl]], wsum)
            return 0

        lax.fori_loop(0, CD // 16, grp, 0)
        return 0

    lax.fori_loop(0, NCHD, chunk, 0)
    pltpu.sync_copy(degbuf, deg_out.at[pl.ds(t * N, N)])


_sc_degree = pl.kernel(
    _sc_deg_body,
    out_type=jax.ShapeDtypeStruct((NT * N,), jnp.float32),
    mesh=_sc_mesh,
    compiler_params=pltpu.CompilerParams(needs_layout_passes=False),
    scratch_types=[
        pltpu.VMEM((N,), jnp.float32),
        pltpu.VMEM((CD,), jnp.float32),
        pltpu.VMEM((CD,), jnp.float32),
        pltpu.VMEM((CD,), jnp.float32),
        pltpu.VMEM((CD,), jnp.float32),
        pltpu.VMEM((CD,), jnp.int32),
    ],
)


def _fin1_body(acc_ref, deg_ref, x_ref, root_ref, b_ref, wt2_ref,
               h_ref, z_ref):
    sacc = acc_ref[0] + acc_ref[1]
    deg = jnp.maximum(jnp.sum(deg_ref[...], axis=1), 1.0)
    o = (sacc / deg[:, None]
         + jnp.dot(x_ref[...], root_ref[...],
                   preferred_element_type=jnp.float32)
         + b_ref[...])
    h = jnp.where(o > 0, o, jnp.exp(o) - 1.0)
    h_ref[...] = h
    z_ref[...] = jnp.dot(h, wt2_ref[...], preferred_element_type=jnp.float32)


def _fin1(acc, degT, xx, root, b2d, wt2):
    return pl.pallas_call(
        _fin1_body,
        grid=(NRB,),
        in_specs=[pl.BlockSpec((NC, RB, OUT), lambda i: (0, i, 0)),
                  pl.BlockSpec((RB, NT), lambda i: (i, 0)),
                  pl.BlockSpec((RB, IN), lambda i: (i, 0)),
                  pl.BlockSpec((IN, OUT), lambda i: (0, 0)),
                  pl.BlockSpec((1, OUT), lambda i: (0, 0)),
                  pl.BlockSpec((IN, K * OUT), lambda i: (0, 0))],
        out_specs=[pl.BlockSpec((RB, OUT), lambda i: (i, 0)),
                   pl.BlockSpec((RB, K * OUT), lambda i: (i, 0))],
        out_shape=[jax.ShapeDtypeStruct((N, OUT), jnp.float32),
                   jax.ShapeDtypeStruct((N, K * OUT), jnp.float32)],
    )(acc, degT, xx, root, b2d, wt2)


def _fin2_body(acc_ref, deg_ref, h1_ref, root_ref, b_ref, h_ref):
    sacc = acc_ref[0] + acc_ref[1]
    deg = jnp.maximum(jnp.sum(deg_ref[...], axis=1), 1.0)
    o = (sacc / deg[:, None]
         + jnp.dot(h1_ref[...], root_ref[...],
                   preferred_element_type=jnp.float32)
         + b_ref[...])
    h_ref[...] = jnp.where(o > 0, o, jnp.exp(o) - 1.0)


def _fin2(acc, degT, h1, root, b2d):
    return pl.pallas_call(
        _fin2_body,
        grid=(NRB,),
        in_specs=[pl.BlockSpec((NC, RB, OUT), lambda i: (0, i, 0)),
                  pl.BlockSpec((RB, NT), lambda i: (i, 0)),
                  pl.BlockSpec((RB, IN), lambda i: (i, 0)),
                  pl.BlockSpec((IN, OUT), lambda i: (0, 0)),
                  pl.BlockSpec((1, OUT), lambda i: (0, 0))],
        out_specs=pl.BlockSpec((RB, OUT), lambda i: (i, 0)),
        out_shape=jax.ShapeDtypeStruct((N, OUT), jnp.float32),
    )(acc, degT, h1, root, b2d)


def kernel(x, edge_index, edge_attr, W1, root1, b1, W2, root2, b2):
    r = E // 128
    src2d = edge_index[0].astype(jnp.int32).reshape(r, 128)
    dst = edge_index[1].astype(jnp.int32)
    p02d = edge_attr[:, 0].reshape(r, 128)
    p12d = edge_attr[:, 1].reshape(r, 128)

    w4, i4 = _basis(src2d, p02d, p12d)
    w4 = w4.reshape(4, E)
    i4 = i4.reshape(4, E)

    wmeta = jnp.pad(w4, ((0, 0), (0, EPAD - E))).reshape(4 * EPAD)
    imeta = jnp.pad(i4, ((0, 0), (0, EPAD - E))).reshape(4 * EPAD)
    dmeta = jnp.pad(dst, (0, EPAD - E))

    zero2 = jnp.zeros((N, OUT), jnp.float32)
    zero1 = jnp.zeros((N,), jnp.float32)

    wt1 = W1.transpose(1, 0, 2).reshape(IN, K * OUT)
    wt2 = W2.transpose(1, 0, 2).reshape(IN, K * OUT)
    b1_2d = b1.reshape(1, OUT)
    b2_2d = b2.reshape(1, OUT)

    deg = _sc_degree(wmeta, dmeta, zero1)
    degT = deg.reshape(NT, N).T

    z1 = _zmat(x, wt1).reshape(N * K, OUT)
    acc1 = _sc_gather_scatter(z1, imeta, wmeta, dmeta, zero2)
    h1, z2 = _fin1(acc1, degT, x, root1, b1_2d, wt2)
    acc2 = _sc_gather_scatter(z2.reshape(N * K, OUT), imeta, wmeta, dmeta,
                              zero2)
    h2 = _fin2(acc2, degT, h1, root2, b2_2d)
    return h2


# speedup vs baseline: 2.2765x; 1.4625x over previous
import jax
import jax.numpy as jnp
from jax import lax
from jax.experimental import pallas as pl
from jax.experimental.pallas import tpu as pltpu
from jax.experimental.pallas import tpu_sc as plsc

N = 10000
E = 160000
IN = 128
OUT = 128
KS = 5
K = KS * KS
NK = N * K
NC = 2
NS = 16
NT = NC * NS
EPW = 5120
EPAD = NT * EPW
C = 32
NCH = EPW // C
CD = 320
NCHD = EPW // CD
NROW = 624
RB = 400
NRB = N // RB


def _basis_body(src_ref, p0_ref, p1_ref, w_ref, i_ref):
    src = src_ref[...]
    v0 = p0_ref[...] * (KS - 1.0)
    v1 = p1_ref[...] * (KS - 1.0)
    b0 = jnp.clip(jnp.floor(v0).astype(jnp.int32), 0, KS - 2)
    b1 = jnp.clip(jnp.floor(v1).astype(jnp.int32), 0, KS - 2)
    f0 = v0 - b0.astype(jnp.float32)
    f1 = v1 - b1.astype(jnp.float32)
    base = src * K + b0 + b1 * KS
    w_ref[0] = (1.0 - f0) * (1.0 - f1)
    w_ref[1] = f0 * (1.0 - f1)
    w_ref[2] = (1.0 - f0) * f1
    w_ref[3] = f0 * f1
    i_ref[0] = base
    i_ref[1] = base + 1
    i_ref[2] = base + KS
    i_ref[3] = base + KS + 1


def _basis(src2d, p02d, p12d):
    r = E // 128
    return pl.pallas_call(
        _basis_body,
        out_shape=[jax.ShapeDtypeStruct((4, r, 128), jnp.float32),
                   jax.ShapeDtypeStruct((4, r, 128), jnp.int32)],
    )(src2d, p02d, p12d)


def _z_body(x_ref, wt_ref, z_ref):
    z_ref[...] = jnp.dot(x_ref[...], wt_ref[...],
                         preferred_element_type=jnp.float32)


def _zmat(xx, wt):
    return pl.pallas_call(
        _z_body,
        grid=(NRB,),
        in_specs=[pl.BlockSpec((RB, IN), lambda i: (i, 0)),
                  pl.BlockSpec((IN, K * OUT), lambda i: (0, 0))],
        out_specs=pl.BlockSpec((RB, K * OUT), lambda i: (i, 0)),
        out_shape=jax.ShapeDtypeStruct((N, K * OUT), jnp.float32),
    )(xx, wt)


_sc_mesh = plsc.VectorSubcoreMesh(core_axis_name="c", subcore_axis_name="s",
                                  num_cores=NC, num_subcores=NS)


def _lane_splat(vec, l):
    idx = jnp.full((16,), l, dtype=jnp.int32)
    return lax.gather(
        vec, idx[:, None],
        dimension_numbers=lax.GatherDimensionNumbers(
            offset_dims=(), collapsed_slice_dims=(0,), start_index_map=(0,)),
        slice_sizes=(1,), mode=lax.GatherScatterMode.PROMISE_IN_BOUNDS)


def _sc_body(z_ref, im_ref, wm_ref, dm_ref, zero2_ref,
             acc_out,
             acc,
             wbufs, ibufs, dbufs, gbufs, mb,
             msems, gsems):
    c = lax.axis_index("c")
    s = lax.axis_index("s")
    t = c * NS + s
    base = s * NROW
    tail = NS * NROW
    moff = t * EPW

    pltpu.sync_copy(zero2_ref.at[pl.ds(base, NROW)], acc.at[pl.ds(base, NROW)])

    @pl.when(s == NS - 1)
    def _():
        pltpu.sync_copy(zero2_ref.at[pl.ds(tail, N - tail)],
                        acc.at[pl.ds(tail, N - tail)])

    plsc.subcore_barrier()

    def issue_meta(j, b):
        pltpu.async_copy(wm_ref.at[pl.ds((moff + j * C) * 4, 4 * C)],
                         wbufs[b], msems[b])
        pltpu.async_copy(im_ref.at[pl.ds((moff + j * C) * 4, 4 * C)],
                         ibufs[b], msems[b])
        pltpu.async_copy(dm_ref.at[pl.ds(moff + j * C, C)],
                         dbufs[b], msems[b])

    def wait_meta(j, b):
        pltpu.make_async_copy(wm_ref.at[pl.ds((moff + j * C) * 4, 4 * C)],
                              wbufs[b], msems[b]).wait()
        pltpu.make_async_copy(im_ref.at[pl.ds((moff + j * C) * 4, 4 * C)],
                              ibufs[b], msems[b]).wait()
        pltpu.make_async_copy(dm_ref.at[pl.ds(moff + j * C, C)],
                              dbufs[b], msems[b]).wait()

    def issue_gathers(b):
        for q in range(4):
            pltpu.async_copy(z_ref.at[ibufs[b].at[pl.ds(q * C, C)]],
                             gbufs[b][q], gsems[b])

    def wait_gathers(b):
        for q in range(4):
            pltpu.make_async_copy(z_ref.at[ibufs[b].at[pl.ds(q * C, C)]],
                                  gbufs[b][q], gsems[b]).wait()

    def compute(b):
        wbuf = wbufs[b]
        gb0, gb1, gb2, gb3 = gbufs[b]

        def grp(g, carry2):
            wv0 = wbuf[pl.ds(0 * C + g * 16, 16)]
            wv1 = wbuf[pl.ds(1 * C + g * 16, 16)]
            wv2 = wbuf[pl.ds(2 * C + g * 16, 16)]
            wv3 = wbuf[pl.ds(3 * C + g * 16, 16)]

            def lane(l, carry3):
                e = g * 16 + l
                s0 = _lane_splat(wv0, l)
                s1 = _lane_splat(wv1, l)
                s2 = _lane_splat(wv2, l)
                s3 = _lane_splat(wv3, l)
                for v in range(OUT // 16):
                    col = pl.ds(v * 16, 16)
                    m = (gb0[e, col] * s0 + gb1[e, col] * s1 +
                         gb2[e, col] * s2 + gb3[e, col] * s3)
                    mb[e, col] = m
                return 0

            lax.fori_loop(0, 16, lane, 0)
            return 0

        lax.fori_loop(0, C // 16, grp, 0)

    issue_meta(0, 0)
    wait_meta(0, 0)
    issue_gathers(0)
    issue_meta(1, 1)

    def step(jj, carry):
        for b in range(2):
            j = jj * 2 + b
            nb = 1 - b

            @pl.when(j + 1 < NCH)
            def _():
                wait_meta(j + 1, nb)
                issue_gathers(nb)

            wait_gathers(b)
            compute(b)
            pltpu.sync_copy(mb, acc.at[dbufs[b]], add=True)

            @pl.when(j + 2 < NCH)
            def _():
                issue_meta(j + 2, b)

        return 0

    lax.fori_loop(0, NCH // 2, step, 0)
    plsc.subcore_barrier()
    pltpu.sync_copy(acc.at[pl.ds(base, NROW)],
                    acc_out.at[c, pl.ds(base, NROW)])

    @pl.when(s == NS - 1)
    def _():
        pltpu.sync_copy(acc.at[pl.ds(tail, N - tail)],
                        acc_out.at[c, pl.ds(tail, N - tail)])


_sc_gather_scatter = pl.kernel(
    _sc_body,
    out_type=jax.ShapeDtypeStruct((NC, N, OUT), jnp.float32),
    mesh=_sc_mesh,
    compiler_params=pltpu.CompilerParams(needs_layout_passes=False),
    scratch_types=[
        pltpu.VMEM_SHARED((N, OUT), jnp.float32),
        [pltpu.VMEM((4 * C,), jnp.float32)] * 2,
        [pltpu.VMEM((4 * C,), jnp.int32)] * 2,
        [pltpu.VMEM((C,), jnp.int32)] * 2,
        [[pltpu.VMEM((C, OUT), jnp.float32)] * 4] * 2,
        pltpu.VMEM((C, OUT), jnp.float32),
        [pltpu.SemaphoreType.DMA] * 2,
        [pltpu.SemaphoreType.DMA] * 2,
    ],
)


def _sc_deg_body(wm_ref, dm_ref, zero1_ref,
                 deg_out,
                 degbuf, wb0, wb1, wb2, wb3, db):
    c = lax.axis_index("c")
    s = lax.axis_index("s")
    t = c * NS + s

    pltpu.sync_copy(zero1_ref, degbuf)
    wbs = (wb0, wb1, wb2, wb3)

    def chunk(j, carry):
        mbase = t * EPW + j * CD
        for q in range(4):
            pltpu.sync_copy(wm_ref.at[pl.ds(q * EPAD + mbase, CD)], wbs[q])
        pltpu.sync_copy(dm_ref.at[pl.ds(mbase, CD)], db)

        def grp(g, carry2):
            sl = pl.ds(g * 16, 16)
            wsum = wb0[sl] + wb1[sl] + wb2[sl] + wb3[sl]
            plsc.addupdate_scatter(degbuf, [db[sl]], wsum)
            return 0

        lax.fori_loop(0, CD // 16, grp, 0)
        return 0

    lax.fori_loop(0, NCHD, chunk, 0)
    pltpu.sync_copy(degbuf, deg_out.at[pl.ds(t * N, N)])


_sc_degree = pl.kernel(
    _sc_deg_body,
    out_type=jax.ShapeDtypeStruct((NT * N,), jnp.float32),
    mesh=_sc_mesh,
    compiler_params=pltpu.CompilerParams(needs_layout_passes=False),
    scratch_types=[
        pltpu.VMEM((N,), jnp.float32),
        pltpu.VMEM((CD,), jnp.float32),
        pltpu.VMEM((CD,), jnp.float32),
        pltpu.VMEM((CD,), jnp.float32),
        pltpu.VMEM((CD,), jnp.float32),
        pltpu.VMEM((CD,), jnp.int32),
    ],
)


def _fin1_body(acc_ref, deg_ref, x_ref, root_ref, b_ref, wt2_ref,
               h_ref, z_ref):
    sacc = acc_ref[0] + acc_ref[1]
    deg = jnp.maximum(jnp.sum(deg_ref[...], axis=1), 1.0)
    o = (sacc / deg[:, None]
         + jnp.dot(x_ref[...], root_ref[...],
                   preferred_element_type=jnp.float32)
         + b_ref[...])
    h = jnp.where(o > 0, o, jnp.exp(o) - 1.0)
    h_ref[...] = h
    z_ref[...] = jnp.dot(h, wt2_ref[...], preferred_element_type=jnp.float32)


def _fin1(acc, degT, xx, root, b2d, wt2):
    return pl.pallas_call(
        _fin1_body,
        grid=(NRB,),
        in_specs=[pl.BlockSpec((NC, RB, OUT), lambda i: (0, i, 0)),
                  pl.BlockSpec((RB, NT), lambda i: (i, 0)),
                  pl.BlockSpec((RB, IN), lambda i: (i, 0)),
                  pl.BlockSpec((IN, OUT), lambda i: (0, 0)),
                  pl.BlockSpec((1, OUT), lambda i: (0, 0)),
                  pl.BlockSpec((IN, K * OUT), lambda i: (0, 0))],
        out_specs=[pl.BlockSpec((RB, OUT), lambda i: (i, 0)),
                   pl.BlockSpec((RB, K * OUT), lambda i: (i, 0))],
        out_shape=[jax.ShapeDtypeStruct((N, OUT), jnp.float32),
                   jax.ShapeDtypeStruct((N, K * OUT), jnp.float32)],
    )(acc, degT, xx, root, b2d, wt2)


def _fin2_body(acc_ref, deg_ref, h1_ref, root_ref, b_ref, h_ref):
    sacc = acc_ref[0] + acc_ref[1]
    deg = jnp.maximum(jnp.sum(deg_ref[...], axis=1), 1.0)
    o = (sacc / deg[:, None]
         + jnp.dot(h1_ref[...], root_ref[...],
                   preferred_element_type=jnp.float32)
         + b_ref[...])
    h_ref[...] = jnp.where(o > 0, o, jnp.exp(o) - 1.0)


def _fin2(acc, degT, h1, root, b2d):
    return pl.pallas_call(
        _fin2_body,
        grid=(NRB,),
        in_specs=[pl.BlockSpec((NC, RB, OUT), lambda i: (0, i, 0)),
                  pl.BlockSpec((RB, NT), lambda i: (i, 0)),
                  pl.BlockSpec((RB, IN), lambda i: (i, 0)),
                  pl.BlockSpec((IN, OUT), lambda i: (0, 0)),
                  pl.BlockSpec((1, OUT), lambda i: (0, 0))],
        out_specs=pl.BlockSpec((RB, OUT), lambda i: (i, 0)),
        out_shape=jax.ShapeDtypeStruct((N, OUT), jnp.float32),
    )(acc, degT, h1, root, b2d)


def kernel(x, edge_index, edge_attr, W1, root1, b1, W2, root2, b2):
    r = E // 128
    src2d = edge_index[0].astype(jnp.int32).reshape(r, 128)
    dst = edge_index[1].astype(jnp.int32)
    p02d = edge_attr[:, 0].reshape(r, 128)
    p12d = edge_attr[:, 1].reshape(r, 128)

    w4, i4 = _basis(src2d, p02d, p12d)
    w4 = w4.reshape(4, E)
    i4 = i4.reshape(4, E)

    w4p = jnp.pad(w4, ((0, 0), (0, EPAD - E)))
    i4p = jnp.pad(i4, ((0, 0), (0, EPAD - E)))
    wI = w4p.reshape(4, NT * NCH, C).transpose(1, 0, 2).reshape(4 * EPAD)
    iI = i4p.reshape(4, NT * NCH, C).transpose(1, 0, 2).reshape(4 * EPAD)
    wflat = w4p.reshape(4 * EPAD)
    dmeta = jnp.pad(dst, (0, EPAD - E))

    zero2 = jnp.zeros((N, OUT), jnp.float32)
    zero1 = jnp.zeros((N,), jnp.float32)

    wt1 = W1.transpose(1, 0, 2).reshape(IN, K * OUT)
    wt2 = W2.transpose(1, 0, 2).reshape(IN, K * OUT)
    b1_2d = b1.reshape(1, OUT)
    b2_2d = b2.reshape(1, OUT)

    deg = _sc_degree(wflat, dmeta, zero1)
    degT = deg.reshape(NT, N).T

    z1 = _zmat(x, wt1).reshape(NK, OUT)
    acc1 = _sc_gather_scatter(z1, iI, wI, dmeta, zero2)
    h1, z2 = _fin1(acc1, degT, x, root1, b1_2d, wt2)
    acc2 = _sc_gather_scatter(z2.reshape(NK, OUT), iI, wI, dmeta, zero2)
    h2 = _fin2(acc2, degT, h1, root2, b2_2d)
    return h2


# speedup vs baseline: 3.9192x; 1.7216x over previous
import jax
import jax.numpy as jnp
from jax import lax
from jax.experimental import pallas as pl
from jax.experimental.pallas import tpu as pltpu
from jax.experimental.pallas import tpu_sc as plsc

N = 10000
E = 160000
IN = 128
OUT = 128
KS = 5
K = KS * KS
NK = N * K
NC = 2
NS = 16
NT = NC * NS
EPW = 5120
EPAD = NT * EPW
C = 64
NCH = EPW // C
CD = 320
NCHD = EPW // CD
NROW = 624
RB = 400
NRB = N // RB


def _basis_body(src_ref, p0_ref, p1_ref, w_ref, i_ref):
    src = src_ref[...]
    v0 = p0_ref[...] * (KS - 1.0)
    v1 = p1_ref[...] * (KS - 1.0)
    b0 = jnp.clip(jnp.floor(v0).astype(jnp.int32), 0, KS - 2)
    b1 = jnp.clip(jnp.floor(v1).astype(jnp.int32), 0, KS - 2)
    f0 = v0 - b0.astype(jnp.float32)
    f1 = v1 - b1.astype(jnp.float32)
    base = src * K + b0 + b1 * KS
    w_ref[0] = (1.0 - f0) * (1.0 - f1)
    w_ref[1] = f0 * (1.0 - f1)
    w_ref[2] = (1.0 - f0) * f1
    w_ref[3] = f0 * f1
    i_ref[0] = base
    i_ref[1] = base + KS


def _basis(src2d, p02d, p12d):
    r = E // 128
    return pl.pallas_call(
        _basis_body,
        out_shape=[jax.ShapeDtypeStruct((4, r, 128), jnp.float32),
                   jax.ShapeDtypeStruct((2, r, 128), jnp.int32)],
    )(src2d, p02d, p12d)


def _pack_z(xh, wtpad_ref):
    cols = []
    for k in range(K):
        a = jnp.dot(xh, wtpad_ref[:, k * OUT:(k + 1) * OUT],
                    preferred_element_type=jnp.float32)
        b = jnp.dot(xh, wtpad_ref[:, (k + 1) * OUT:(k + 2) * OUT],
                    preferred_element_type=jnp.float32)
        ai = lax.bitcast_convert_type(a.astype(jnp.bfloat16),
                                      jnp.uint16).astype(jnp.int32)
        bi = lax.bitcast_convert_type(b.astype(jnp.bfloat16),
                                      jnp.uint16).astype(jnp.int32)
        cols.append(ai | (bi << 16))
    return jnp.concatenate(cols, axis=1)


def _z_body(x_ref, wt_ref, z_ref):
    z_ref[...] = _pack_z(x_ref[...], wt_ref)


def _zmat(xx, wtpad):
    return pl.pallas_call(
        _z_body,
        grid=(NRB,),
        in_specs=[pl.BlockSpec((RB, IN), lambda i: (i, 0)),
                  pl.BlockSpec((IN, (K + 1) * OUT), lambda i: (0, 0))],
        out_specs=pl.BlockSpec((RB, K * OUT), lambda i: (i, 0)),
        out_shape=jax.ShapeDtypeStruct((N, K * OUT), jnp.int32),
    )(xx, wtpad)


_sc_mesh = plsc.VectorSubcoreMesh(core_axis_name="c", subcore_axis_name="s",
                                  num_cores=NC, num_subcores=NS)


def _lane_splat(vec, l):
    idx = jnp.full((16,), l, dtype=jnp.int32)
    return lax.gather(
        vec, idx[:, None],
        dimension_numbers=lax.GatherDimensionNumbers(
            offset_dims=(), collapsed_slice_dims=(0,), start_index_map=(0,)),
        slice_sizes=(1,), mode=lax.GatherScatterMode.PROMISE_IN_BOUNDS)


def _sc_body(z_ref, im_ref, wm_ref, dm_ref, zero2_ref,
             acc_out,
             acc,
             wbufs, ibufs, dbufs, gbufs, mb,
             msems, gsems):
    c = lax.axis_index("c")
    s = lax.axis_index("s")
    t = c * NS + s
    base = s * NROW
    tail = NS * NROW
    moff = t * EPW

    pltpu.sync_copy(zero2_ref.at[pl.ds(base, NROW)], acc.at[pl.ds(base, NROW)])

    @pl.when(s == NS - 1)
    def _():
        pltpu.sync_copy(zero2_ref.at[pl.ds(tail, N - tail)],
                        acc.at[pl.ds(tail, N - tail)])

    plsc.subcore_barrier()

    def issue_meta(j, b):
        pltpu.async_copy(wm_ref.at[pl.ds((moff + j * C) * 4, 4 * C)],
                         wbufs[b], msems[b])
        pltpu.async_copy(im_ref.at[pl.ds((moff + j * C) * 2, 2 * C)],
                         ibufs[b], msems[b])
        pltpu.async_copy(dm_ref.at[pl.ds(moff + j * C, C)],
                         dbufs[b], msems[b])

    def wait_meta(j, b):
        pltpu.make_async_copy(wm_ref.at[pl.ds((moff + j * C) * 4, 4 * C)],
                              wbufs[b], msems[b]).wait()
        pltpu.make_async_copy(im_ref.at[pl.ds((moff + j * C) * 2, 2 * C)],
                              ibufs[b], msems[b]).wait()
        pltpu.make_async_copy(dm_ref.at[pl.ds(moff + j * C, C)],
                              dbufs[b], msems[b]).wait()

    def issue_gathers(b):
        for q in range(2):
            pltpu.async_copy(z_ref.at[ibufs[b].at[pl.ds(q * C, C)]],
                             gbufs[b][q], gsems[b])

    def wait_gathers(b):
        for q in range(2):
            pltpu.make_async_copy(z_ref.at[ibufs[b].at[pl.ds(q * C, C)]],
                                  gbufs[b][q], gsems[b]).wait()

    himask = jnp.full((16,), -65536, jnp.int32)

    def compute(b):
        wbuf = wbufs[b]
        gba, gbb = gbufs[b]

        def grp(g, carry2):
            wv0 = wbuf[pl.ds(0 * C + g * 16, 16)]
            wv1 = wbuf[pl.ds(1 * C + g * 16, 16)]
            wv2 = wbuf[pl.ds(2 * C + g * 16, 16)]
            wv3 = wbuf[pl.ds(3 * C + g * 16, 16)]

            def lane(l, carry3):
                e = g * 16 + l
                s0 = _lane_splat(wv0, l)
                s1 = _lane_splat(wv1, l)
                s2 = _lane_splat(wv2, l)
                s3 = _lane_splat(wv3, l)
                for v in range(OUT // 16):
                    col = pl.ds(v * 16, 16)
                    ga = gba[e, col]
                    gb = gbb[e, col]
                    m = (plsc.bitcast(ga << 16, jnp.float32) * s0 +
                         plsc.bitcast(ga & himask, jnp.float32) * s1 +
                         plsc.bitcast(gb << 16, jnp.float32) * s2 +
                         plsc.bitcast(gb & himask, jnp.float32) * s3)
                    mb[e, col] = m
                return 0

            lax.fori_loop(0, 16, lane, 0)
            return 0

        lax.fori_loop(0, C // 16, grp, 0)

    issue_meta(0, 0)
    wait_meta(0, 0)
    issue_gathers(0)
    issue_meta(1, 1)

    def step(jj, carry):
        for b in range(2):
            j = jj * 2 + b
            nb = 1 - b

            @pl.when(j + 1 < NCH)
            def _():
                wait_meta(j + 1, nb)
                issue_gathers(nb)

            wait_gathers(b)
            compute(b)
            pltpu.sync_copy(mb, acc.at[dbufs[b]], add=True)

            @pl.when(j + 2 < NCH)
            def _():
                issue_meta(j + 2, b)

        return 0

    lax.fori_loop(0, NCH // 2, step, 0)
    plsc.subcore_barrier()
    pltpu.sync_copy(acc.at[pl.ds(base, NROW)],
                    acc_out.at[c, pl.ds(base, NROW)])

    @pl.when(s == NS - 1)
    def _():
        pltpu.sync_copy(acc.at[pl.ds(tail, N - tail)],
                        acc_out.at[c, pl.ds(tail, N - tail)])


_sc_gather_scatter = pl.kernel(
    _sc_body,
    out_type=jax.ShapeDtypeStruct((NC, N, OUT), jnp.float32),
    mesh=_sc_mesh,
    compiler_params=pltpu.CompilerParams(needs_layout_passes=False),
    scratch_types=[
        pltpu.VMEM_SHARED((N, OUT), jnp.float32),
        [pltpu.VMEM((4 * C,), jnp.float32)] * 2,
        [pltpu.VMEM((2 * C,), jnp.int32)] * 2,
        [pltpu.VMEM((C,), jnp.int32)] * 2,
        [[pltpu.VMEM((C, OUT), jnp.int32)] * 2] * 2,
        pltpu.VMEM((C, OUT), jnp.float32),
        [pltpu.SemaphoreType.DMA] * 2,
        [pltpu.SemaphoreType.DMA] * 2,
    ],
)


def _sc_deg_body(wm_ref, dm_ref, zero1_ref,
                 deg_out,
                 degbuf, wb0, wb1, wb2, wb3, db):
    c = lax.axis_index("c")
    s = lax.axis_index("s")
    t = c * NS + s

    pltpu.sync_copy(zero1_ref, degbuf)
    wbs = (wb0, wb1, wb2, wb3)

    def chunk(j, carry):
        mbase = t * EPW + j * CD
        for q in range(4):
            pltpu.sync_copy(wm_ref.at[pl.ds(q * EPAD + mbase, CD)], wbs[q])
        pltpu.sync_copy(dm_ref.at[pl.ds(mbase, CD)], db)

        def grp(g, carry2):
            sl = pl.ds(g * 16, 16)
            wsum = wb0[sl] + wb1[sl] + wb2[sl] + wb3[sl]
            plsc.addupdate_scatter(degbuf, [db[sl]], wsum)
            return 0

        lax.fori_loop(0, CD // 16, grp, 0)
        return 0

    lax.fori_loop(0, NCHD, chunk, 0)
    pltpu.sync_copy(degbuf, deg_out.at[pl.ds(t * N, N)])


_sc_degree = pl.kernel(
    _sc_deg_body,
    out_type=jax.ShapeDtypeStruct((NT * N,), jnp.float32),
    mesh=_sc_mesh,
    compiler_params=pltpu.CompilerParams(needs_layout_passes=False),
    scratch_types=[
        pltpu.VMEM((N,), jnp.float32),
        pltpu.VMEM((CD,), jnp.float32),
        pltpu.VMEM((CD,), jnp.float32),
        pltpu.VMEM((CD,), jnp.float32),
        pltpu.VMEM((CD,), jnp.float32),
        pltpu.VMEM((CD,), jnp.int32),
    ],
)


def _fin1_body(acc_ref, deg_ref, x_ref, root_ref, b_ref, wt2_ref,
               h_ref, z_ref):
    sacc = acc_ref[0] + acc_ref[1]
    deg = jnp.maximum(jnp.sum(deg_ref[...], axis=1), 1.0)
    o = (sacc / deg[:, None]
         + jnp.dot(x_ref[...], root_ref[...],
                   preferred_element_type=jnp.float32)
         + b_ref[...])
    h = jnp.where(o > 0, o, jnp.exp(o) - 1.0)
    h_ref[...] = h
    z_ref[...] = _pack_z(h, wt2_ref)


def _fin1(acc, degT, xx, root, b2d, wtpad2):
    return pl.pallas_call(
        _fin1_body,
        grid=(NRB,),
        in_specs=[pl.BlockSpec((NC, RB, OUT), lambda i: (0, i, 0)),
                  pl.BlockSpec((RB, NT), lambda i: (i, 0)),
                  pl.BlockSpec((RB, IN), lambda i: (i, 0)),
                  pl.BlockSpec((IN, OUT), lambda i: (0, 0)),
                  pl.BlockSpec((1, OUT), lambda i: (0, 0)),
                  pl.BlockSpec((IN, (K + 1) * OUT), lambda i: (0, 0))],
        out_specs=[pl.BlockSpec((RB, OUT), lambda i: (i, 0)),
                   pl.BlockSpec((RB, K * OUT), lambda i: (i, 0))],
        out_shape=[jax.ShapeDtypeStruct((N, OUT), jnp.float32),
                   jax.ShapeDtypeStruct((N, K * OUT), jnp.int32)],
    )(acc, degT, xx, root, b2d, wtpad2)


def _fin2_body(acc_ref, deg_ref, h1_ref, root_ref, b_ref, h_ref):
    sacc = acc_ref[0] + acc_ref[1]
    deg = jnp.maximum(jnp.sum(deg_ref[...], axis=1), 1.0)
    o = (sacc / deg[:, None]
         + jnp.dot(h1_ref[...], root_ref[...],
                   preferred_element_type=jnp.float32)
         + b_ref[...])
    h_ref[...] = jnp.where(o > 0, o, jnp.exp(o) - 1.0)


def _fin2(acc, degT, h1, root, b2d):
    return pl.pallas_call(
        _fin2_body,
        grid=(NRB,),
        in_specs=[pl.BlockSpec((NC, RB, OUT), lambda i: (0, i, 0)),
                  pl.BlockSpec((RB, NT), lambda i: (i, 0)),
                  pl.BlockSpec((RB, IN), lambda i: (i, 0)),
                  pl.BlockSpec((IN, OUT), lambda i: (0, 0)),
                  pl.BlockSpec((1, OUT), lambda i: (0, 0))],
        out_specs=pl.BlockSpec((RB, OUT), lambda i: (i, 0)),
        out_shape=jax.ShapeDtypeStruct((N, OUT), jnp.float32),
    )(acc, degT, h1, root, b2d)


def kernel(x, edge_index, edge_attr, W1, root1, b1, W2, root2, b2):
    r = E // 128
    src2d = edge_index[0].astype(jnp.int32).reshape(r, 128)
    dst = edge_index[1].astype(jnp.int32)
    p02d = edge_attr[:, 0].reshape(r, 128)
    p12d = edge_attr[:, 1].reshape(r, 128)

    w4, i2 = _basis(src2d, p02d, p12d)
    w4 = w4.reshape(4, E)
    i2 = i2.reshape(2, E)

    w4p = jnp.pad(w4, ((0, 0), (0, EPAD - E)))
    i2p = jnp.pad(i2, ((0, 0), (0, EPAD - E)))
    wI = w4p.reshape(4, NT * NCH, C).transpose(1, 0, 2).reshape(4 * EPAD)
    iI = i2p.reshape(2, NT * NCH, C).transpose(1, 0, 2).reshape(2 * EPAD)
    wflat = w4p.reshape(4 * EPAD)
    dmeta = jnp.pad(dst, (0, EPAD - E))

    zero2 = jnp.zeros((N, OUT), jnp.float32)
    zero1 = jnp.zeros((N,), jnp.float32)

    wt1 = jnp.pad(W1.transpose(1, 0, 2).reshape(IN, K * OUT),
                  ((0, 0), (0, OUT)))
    wt2 = jnp.pad(W2.transpose(1, 0, 2).reshape(IN, K * OUT),
                  ((0, 0), (0, OUT)))
    b1_2d = b1.reshape(1, OUT)
    b2_2d = b2.reshape(1, OUT)

    deg = _sc_degree(wflat, dmeta, zero1)
    degT = deg.reshape(NT, N).T

    z1 = _zmat(x, wt1).reshape(NK, OUT)
    acc1 = _sc_gather_scatter(z1, iI, wI, dmeta, zero2)
    h1, z2 = _fin1(acc1, degT, x, root1, b1_2d, wt2)
    acc2 = _sc_gather_scatter(z2.reshape(NK, OUT), iI, wI, dmeta, zero2)
    h2 = _fin2(acc2, degT, h1, root2, b2_2d)
    return h2
